# baseline (speedup 1.0000x reference)
import jax
import jax.numpy as jnp
from jax import lax
from jax.experimental import pallas as pl
from jax.experimental.pallas import tpu as pltpu
from jax.experimental.pallas import tpu_sc as plsc

B = 16384
NC, NS, L = 2, 16, 16
NW = NC * NS
CH = B // NW
NV = CH // L

LOG_SCALE_MEAN = 2.5
SQRT2 = 1.4142135623730951
LOG10_2 = 0.30102999566398119521
INV_LN10 = 0.43429448190325182765


_P0 = 4.971411304351899e-06 - LOG_SCALE_MEAN
_P1 = 0.4339324544622425
_P2 = -0.2126736127194115
_P3 = 0.12326284224157259
_P4 = -0.0564396938266918
_P5 = 0.012945782257621382


def _log_scale_norm(x):
    bits = lax.bitcast_convert_type(x, jnp.int32)
    e = jnp.right_shift(bits, 23) - 127
    t = lax.bitcast_convert_type((bits & 0x007FFFFF) | 0x3F800000,
                                 jnp.float32) - 1.0
    p = _P0 + t * (_P1 + t * (_P2 + t * (_P3 + t * (_P4 + t * _P5))))
    return e.astype(jnp.float32) * LOG10_2 + p


def _sc_body(dom_hbm, sc_hbm, rn_hbm, bg_hbm, out_hbm,
             dom_v, sc_v, rn_v, bg_v, out_v, sem, osem):
    wid = lax.axis_index("s") * NC + lax.axis_index("c")
    base = wid * CH
    copies = [
        pltpu.make_async_copy(dom_hbm.at[pl.ds(base, CH)], dom_v, sem),
        pltpu.make_async_copy(sc_hbm.at[pl.ds(base, CH)], sc_v, sem),
        pltpu.make_async_copy(rn_hbm.at[pl.ds(base, CH)], rn_v, sem),
        pltpu.make_async_copy(bg_hbm.at[pl.ds(base, CH)], bg_v, sem),
    ]
    for c in copies:
        c.start()
    for c in copies:
        c.wait()
    half = CH // 2
    out_copies = [
        pltpu.make_async_copy(out_v.at[:, pl.ds(h * half, half)],
                              out_hbm.at[:, pl.ds(base + h * half, half)],
                              osem)
        for h in range(2)
    ]
    for i in range(NV):
        sl = pl.ds(i * L, L)
        dom = dom_v[sl]
        cs = jnp.maximum(sc_v[sl], 1e-6)
        inv = 1.0 / cs
        cols = (
            jnp.where(dom == 0, 1.0, 0.0),
            jnp.where(dom == 1, 1.0, 0.0),
            jnp.where(dom == 2, 1.0, 0.0),
            _log_scale_norm(cs),
            rn_v[sl] * inv,
            bg_v[sl] * inv,
        )
        for c, val in enumerate(cols):
            out_v[c, pl.ds(i * L, L)] = val
        if i == NV // 2 - 1:
            out_copies[0].start()
    out_copies[1].start()
    out_copies[0].wait()
    out_copies[1].wait()


@jax.jit
def kernel(domain, scale, read_noise, background):
    run = pl.kernel(
        _sc_body,
        out_type=jax.ShapeDtypeStruct((6, B), jnp.float32),
        mesh=plsc.VectorSubcoreMesh(
            core_axis_name="c", subcore_axis_name="s",
            num_cores=NC, num_subcores=NS),
        scratch_types=[
            pltpu.VMEM((CH,), jnp.int32),
            pltpu.VMEM((CH,), jnp.float32),
            pltpu.VMEM((CH,), jnp.float32),
            pltpu.VMEM((CH,), jnp.float32),
            pltpu.VMEM((6, CH), jnp.float32),
            pltpu.SemaphoreType.DMA,
            pltpu.SemaphoreType.DMA,
        ],
        compiler_params=pltpu.CompilerParams(needs_layout_passes=False),
    )
    return run(domain, scale, read_noise, background).T


# speedup vs baseline: 1.0822x; 1.0822x over previous
import jax
import jax.numpy as jnp
from jax import lax
from jax.experimental import pallas as pl
from jax.experimental.pallas import tpu as pltpu
from jax.experimental.pallas import tpu_sc as plsc

B = 16384
NC, NS, L = 1, 16, 16
NW = NC * NS
CH = B // NW
NV = CH // L

LOG_SCALE_MEAN = 2.5
SQRT2 = 1.4142135623730951
LOG10_2 = 0.30102999566398119521
INV_LN10 = 0.43429448190325182765


_P0 = 4.971411304351899e-06 - LOG_SCALE_MEAN
_P1 = 0.4339324544622425
_P2 = -0.2126736127194115
_P3 = 0.12326284224157259
_P4 = -0.0564396938266918
_P5 = 0.012945782257621382


def _log_scale_norm(x):
    bits = lax.bitcast_convert_type(x, jnp.int32)
    e = jnp.right_shift(bits, 23) - 127
    t = lax.bitcast_convert_type((bits & 0x007FFFFF) | 0x3F800000,
                                 jnp.float32) - 1.0
    p = _P0 + t * (_P1 + t * (_P2 + t * (_P3 + t * (_P4 + t * _P5))))
    return e.astype(jnp.float32) * LOG10_2 + p


def _sc_body(dom_hbm, sc_hbm, rn_hbm, bg_hbm, out_hbm,
             dom_v, sc_v, rn_v, bg_v, out_v, sem, osem):
    wid = lax.axis_index("s") * NC + lax.axis_index("c")
    base = wid * CH
    copies = [
        pltpu.make_async_copy(dom_hbm.at[pl.ds(base, CH)], dom_v, sem),
        pltpu.make_async_copy(sc_hbm.at[pl.ds(base, CH)], sc_v, sem),
        pltpu.make_async_copy(rn_hbm.at[pl.ds(base, CH)], rn_v, sem),
        pltpu.make_async_copy(bg_hbm.at[pl.ds(base, CH)], bg_v, sem),
    ]
    for c in copies:
        c.start()
    for c in copies:
        c.wait()
    half = CH // 2
    out_copies = [
        pltpu.make_async_copy(out_v.at[:, pl.ds(h * half, half)],
                              out_hbm.at[:, pl.ds(base + h * half, half)],
                              osem)
        for h in range(2)
    ]
    def _compute(i):
        sl = pl.ds(i, L)
        dom = dom_v[sl]
        cs = jnp.maximum(sc_v[sl], 1e-6)
        inv = 1.0 / cs
        cols = (
            jnp.where(dom == 0, 1.0, 0.0),
            jnp.where(dom == 1, 1.0, 0.0),
            jnp.where(dom == 2, 1.0, 0.0),
            _log_scale_norm(cs),
            rn_v[sl] * inv,
            bg_v[sl] * inv,
        )
        for c, val in enumerate(cols):
            out_v[c, pl.ds(i, L)] = val

    @plsc.parallel_loop(0, CH // 2, step=L, unroll=2)
    def _first_half(i):
        _compute(i)

    out_copies[0].start()

    @plsc.parallel_loop(CH // 2, CH, step=L, unroll=2)
    def _second_half(i):
        _compute(i)

    out_copies[1].start()
    out_copies[0].wait()
    out_copies[1].wait()


@jax.jit
def kernel(domain, scale, read_noise, background):
    run = pl.kernel(
        _sc_body,
        out_type=jax.ShapeDtypeStruct((6, B), jnp.float32),
        mesh=plsc.VectorSubcoreMesh(
            core_axis_name="c", subcore_axis_name="s",
            num_cores=NC, num_subcores=NS),
        scratch_types=[
            pltpu.VMEM((CH,), jnp.int32),
            pltpu.VMEM((CH,), jnp.float32),
            pltpu.VMEM((CH,), jnp.float32),
            pltpu.VMEM((CH,), jnp.float32),
            pltpu.VMEM((6, CH), jnp.float32),
            pltpu.SemaphoreType.DMA,
            pltpu.SemaphoreType.DMA,
        ],
        compiler_params=pltpu.CompilerParams(needs_layout_passes=False),
    )
    return run(domain, scale, read_noise, background).T
